# baseline (speedup 1.0000x reference)
import functools

import jax
import jax.numpy as jnp
from jax import lax
from jax.experimental import pallas as pl
from jax.experimental.pallas import tpu as pltpu
from jax.experimental.pallas import tpu_sc as plsc

NUM_EXPERTS = 8
TOP_K = 2
HIDDEN = 2048
TOKENS = 2048
ROWS = NUM_EXPERTS * TOKENS


MM_GRID = 8
MM_TBLK = TOKENS // MM_GRID


def _mm_body(w_ref, h_ref, lt_ref):
    lt_ref[...] = jax.lax.dot_general(
        w_ref[...], h_ref[...], (((1,), (1,)), ((), ())),
        preferred_element_type=jnp.float32)


def _tc_logits(hidden_states, W):
    return pl.pallas_call(
        _mm_body,
        grid=(MM_GRID,),
        in_specs=[
            pl.BlockSpec((NUM_EXPERTS, HIDDEN), lambda i: (0, 0)),
            pl.BlockSpec((MM_TBLK, HIDDEN), lambda i: (i, 0)),
        ],
        out_specs=pl.BlockSpec((NUM_EXPERTS, MM_TBLK), lambda i: (0, i)),
        out_shape=jax.ShapeDtypeStruct((NUM_EXPERTS, TOKENS), jnp.float32),
    )(W, hidden_states)


IDX_GRID = 16
IDX_RBLK = ROWS // IDX_GRID


def _idx_body(idx_ref):
    i = pl.program_id(0)
    ridx = jax.lax.broadcasted_iota(jnp.int32, (IDX_RBLK, HIDDEN), 0)
    idx_ref[...] = (i * IDX_RBLK) % TOKENS + ridx


def _tc_indices():
    return pl.pallas_call(
        _idx_body,
        grid=(IDX_GRID,),
        in_specs=[],
        out_specs=pl.BlockSpec((IDX_RBLK, HIDDEN), lambda i: (i, 0)),
        out_shape=jax.ShapeDtypeStruct((ROWS, HIDDEN), jnp.int32),
    )()


_SC_INFO = plsc.get_sparse_core_info()
_NC, _NS, _L = _SC_INFO.num_cores, _SC_INFO.num_subcores, _SC_INFO.num_lanes
_STRIPE = 128
_NACT = TOKENS // _STRIPE
_NEG = jnp.float32(-3.0e38)


def _sc_scores_body(lt_hbm, out_hbm, lg_v, sc_v, sem):
    wid = lax.axis_index("s") * _NC + lax.axis_index("c")

    @pl.when(wid < _NACT)
    def _():
        base = wid * _STRIPE
        pltpu.sync_copy(lt_hbm.at[:, pl.ds(base, _STRIPE)], lg_v)
        for g in range(_STRIPE // _L):
            sl = pl.ds(g * _L, _L)
            l = [lg_v[e, sl] for e in range(NUM_EXPERTS)]
            m1 = l[0]
            for e in range(1, NUM_EXPERTS):
                m1 = jnp.maximum(m1, l[e])
            i1 = jnp.full((_L,), NUM_EXPERTS, dtype=jnp.int32)
            for e in range(NUM_EXPERTS - 1, -1, -1):
                i1 = jnp.where(l[e] == m1, jnp.int32(e), i1)
            masked = [jnp.where(i1 == e, _NEG, l[e])
                      for e in range(NUM_EXPERTS)]
            m2 = masked[0]
            for e in range(1, NUM_EXPERTS):
                m2 = jnp.maximum(m2, masked[e])
            i2 = jnp.full((_L,), NUM_EXPERTS, dtype=jnp.int32)
            for e in range(NUM_EXPERTS - 1, -1, -1):
                i2 = jnp.where(masked[e] == m2, jnp.int32(e), i2)
            for e in range(NUM_EXPERTS):
                keep = (i1 == e) | (i2 == e)
                sig = 1.0 / (1.0 + jnp.exp(-l[e]))
                sc_v[e, sl] = jnp.where(keep, sig, jnp.float32(0.0))
        pltpu.sync_copy(sc_v, out_hbm.at[:, pl.ds(base, _STRIPE)])


_sc_scores = pl.kernel(
    _sc_scores_body,
    mesh=plsc.VectorSubcoreMesh(core_axis_name="c", subcore_axis_name="s"),
    out_type=jax.ShapeDtypeStruct((NUM_EXPERTS, TOKENS), jnp.float32),
    scratch_types=[
        pltpu.VMEM((NUM_EXPERTS, _STRIPE), jnp.float32),
        pltpu.VMEM((NUM_EXPERTS, _STRIPE), jnp.float32),
        pltpu.SemaphoreType.DMA,
    ],
)


def kernel(hidden_states, W):
    logits_t = _tc_logits(hidden_states, W)
    scores = _sc_scores(logits_t)
    indices = _tc_indices()
    probs = scores.reshape(-1, 1)
    return (scores, indices, probs)


# speedup vs baseline: 1.4003x; 1.4003x over previous
import jax
import jax.numpy as jnp
from jax.experimental import pallas as pl

NUM_EXPERTS = 8
TOP_K = 2
HIDDEN = 2048
TOKENS = 2048
ROWS = NUM_EXPERTS * TOKENS

GRID = 8
TBLK = TOKENS // GRID
RBLK = ROWS // GRID


def _body(w_ref, h_ref, scores_ref, idx_ref):
    i = pl.program_id(0)
    lt = jax.lax.dot_general(
        w_ref[...], h_ref[...], (((1,), (1,)), ((), ())),
        preferred_element_type=jnp.float32)
    eidx = jax.lax.broadcasted_iota(jnp.int32, lt.shape, 0)
    m1 = jnp.max(lt, axis=0, keepdims=True)
    i1 = jnp.min(jnp.where(lt == m1, eidx, NUM_EXPERTS), axis=0, keepdims=True)
    masked = jnp.where(eidx == i1, -jnp.inf, lt)
    m2 = jnp.max(masked, axis=0, keepdims=True)
    i2 = jnp.min(jnp.where(masked == m2, eidx, NUM_EXPERTS), axis=0,
                 keepdims=True)
    keep = (eidx == i1) | (eidx == i2)
    scores_ref[...] = jnp.where(keep, jax.nn.sigmoid(lt), 0.0)
    ridx = jax.lax.broadcasted_iota(jnp.int32, (RBLK, HIDDEN), 0)
    idx_ref[...] = (i * RBLK) % TOKENS + ridx


def kernel(hidden_states, W):
    scores, indices = pl.pallas_call(
        _body,
        grid=(GRID,),
        in_specs=[
            pl.BlockSpec((NUM_EXPERTS, HIDDEN), lambda i: (0, 0)),
            pl.BlockSpec((TBLK, HIDDEN), lambda i: (i, 0)),
        ],
        out_specs=[
            pl.BlockSpec((NUM_EXPERTS, TBLK), lambda i: (0, i)),
            pl.BlockSpec((RBLK, HIDDEN), lambda i: (i, 0)),
        ],
        out_shape=[
            jax.ShapeDtypeStruct((NUM_EXPERTS, TOKENS), jnp.float32),
            jax.ShapeDtypeStruct((ROWS, HIDDEN), jnp.int32),
        ],
    )(W, hidden_states)
    probs = scores.reshape(-1, 1)
    return (scores, indices, probs)
